# Initial kernel scaffold; baseline (speedup 1.0000x reference)
import jax
import jax.numpy as jnp
from jax.experimental import pallas as pl


def kernel(x, table):
    raise NotImplementedError("write your pallas kernel here")


# speedup vs baseline: 2.4270x; 2.4270x over previous
import jax
import jax.numpy as jnp
from jax.experimental import pallas as pl
from jax.experimental.pallas import tpu as pltpu
from jax.experimental.pallas import tpu_sc as plsc

EMBED_DIM = 16
WINDOW = 512


def kernel(x, table):
    batch, seq = x.shape
    num_indices = batch * seq
    indices = x.reshape((1, num_indices)).astype(jnp.int32)

    mesh = plsc.VectorSubcoreMesh(core_axis_name="c", subcore_axis_name="s")

    @pl.kernel(
        out_type=jax.ShapeDtypeStruct((num_indices, EMBED_DIM), table.dtype),
        mesh=mesh,
        compiler_params=pltpu.CompilerParams(use_tc_tiling_on_sc=False),
    )
    def gather_kernel(table_hbm, i_hbm, o_hbm):
        def body(i_vmem, o_vmem):
            pltpu.sync_copy(table_hbm.at[i_vmem.at[0]], o_vmem)

        pltpu.emit_pipeline(
            body,
            grid=(num_indices // WINDOW,),
            in_specs=[pl.BlockSpec((1, WINDOW), index_map=lambda i: (0, i))],
            out_specs=[pl.BlockSpec((WINDOW, EMBED_DIM), index_map=lambda i: (i, 0))],
            core_axis_name=("c", "s"),
            dimension_semantics=(pltpu.PARALLEL,),
        )(i_hbm, o_hbm)

    out = gather_kernel(table, indices)
    return out.reshape(batch, seq, EMBED_DIM)


# speedup vs baseline: 2.5316x; 1.0431x over previous
import jax
import jax.numpy as jnp
from jax.experimental import pallas as pl
from jax.experimental.pallas import tpu as pltpu
from jax.experimental.pallas import tpu_sc as plsc

EMBED_DIM = 16
WINDOW = 2048


def kernel(x, table):
    batch, seq = x.shape
    num_indices = batch * seq
    indices = x.reshape((1, num_indices)).astype(jnp.int32)

    mesh = plsc.VectorSubcoreMesh(core_axis_name="c", subcore_axis_name="s")

    @pl.kernel(
        out_type=jax.ShapeDtypeStruct((num_indices, EMBED_DIM), table.dtype),
        mesh=mesh,
        compiler_params=pltpu.CompilerParams(use_tc_tiling_on_sc=False),
    )
    def gather_kernel(table_hbm, i_hbm, o_hbm):
        def body(i_vmem, o_vmem):
            pltpu.sync_copy(table_hbm.at[i_vmem.at[0]], o_vmem)

        pltpu.emit_pipeline(
            body,
            grid=(num_indices // WINDOW,),
            in_specs=[pl.BlockSpec((1, WINDOW), index_map=lambda i: (0, i))],
            out_specs=[pl.BlockSpec((WINDOW, EMBED_DIM), index_map=lambda i: (i, 0))],
            core_axis_name=("c", "s"),
            dimension_semantics=(pltpu.PARALLEL,),
        )(i_hbm, o_hbm)

    out = gather_kernel(table, indices)
    return out.reshape(batch, seq, EMBED_DIM)


# speedup vs baseline: 2.5672x; 1.0140x over previous
import jax
import jax.numpy as jnp
from jax import lax
from jax.experimental import pallas as pl
from jax.experimental.pallas import tpu as pltpu
from jax.experimental.pallas import tpu_sc as plsc

EMBED_DIM = 16
NUM_WORKERS = 32
WINDOW = 1024
NBUF = 4


def kernel(x, table):
    batch, seq = x.shape
    num_indices = batch * seq
    indices = x.reshape((num_indices,)).astype(jnp.int32)

    rows_per_worker = num_indices // NUM_WORKERS
    chunks = rows_per_worker // WINDOW
    outer = chunks // NBUF

    mesh = plsc.VectorSubcoreMesh(core_axis_name="c", subcore_axis_name="s")

    @pl.kernel(
        out_type=jax.ShapeDtypeStruct((num_indices, EMBED_DIM), table.dtype),
        mesh=mesh,
        compiler_params=pltpu.CompilerParams(use_tc_tiling_on_sc=False),
        scratch_types=[
            pltpu.VMEM((NBUF, WINDOW), jnp.int32),
            pltpu.VMEM((NBUF, WINDOW, EMBED_DIM), jnp.float32),
            pltpu.SemaphoreType.DMA((NBUF,)),
            pltpu.SemaphoreType.DMA((NBUF,)),
            pltpu.SemaphoreType.DMA((NBUF,)),
        ],
    )
    def gather_kernel(table_hbm, i_hbm, o_hbm, idx_v, rows_v, sem_i, sem_g, sem_o):
        wid = lax.axis_index("s") * 2 + lax.axis_index("c")
        wbase = wid * rows_per_worker

        def start_idx(b, c):
            pltpu.async_copy(i_hbm.at[pl.ds(wbase + c * WINDOW, WINDOW)],
                             idx_v.at[b], sem_i.at[b])

        def wait_idx(b):
            pltpu.make_async_copy(i_hbm.at[pl.ds(0, WINDOW)],
                                  idx_v.at[b], sem_i.at[b]).wait()

        def start_gather(b):
            pltpu.async_copy(table_hbm.at[idx_v.at[b]], rows_v.at[b], sem_g.at[b])

        def wait_gather(b):
            pltpu.make_async_copy(table_hbm.at[idx_v.at[b]],
                                  rows_v.at[b], sem_g.at[b]).wait()

        def start_out(b, c):
            pltpu.async_copy(rows_v.at[b],
                             o_hbm.at[pl.ds(wbase + c * WINDOW, WINDOW)],
                             sem_o.at[b])

        def wait_out(b):
            pltpu.make_async_copy(rows_v.at[b],
                                  o_hbm.at[pl.ds(0, WINDOW)], sem_o.at[b]).wait()

        for b in range(NBUF):
            start_idx(b, b)
        for b in range(NBUF):
            wait_idx(b)
            start_gather(b)

        @pl.loop(0, outer - 1)
        def _(o):
            cbase = o * NBUF
            for b in range(NBUF):
                wait_gather(b)
                start_out(b, cbase + b)
                start_idx(b, cbase + NBUF + b)
            for b in range(NBUF):
                wait_out(b)
                wait_idx(b)
                start_gather(b)

        for b in range(NBUF):
            wait_gather(b)
            start_out(b, (outer - 1) * NBUF + b)
        for b in range(NBUF):
            wait_out(b)

    out = gather_kernel(table, indices)
    return out.reshape(batch, seq, EMBED_DIM)


# speedup vs baseline: 4.0490x; 1.5772x over previous
import jax
import jax.numpy as jnp
from jax import lax
from jax.experimental import pallas as pl
from jax.experimental.pallas import tpu as pltpu
from jax.experimental.pallas import tpu_sc as plsc

EMBED_DIM = 16
PACK = 128 // EMBED_DIM
NUM_WORKERS = 32
WINDOW = 512
NBUF = 4


def kernel(x, table):
    batch, seq = x.shape
    num_indices = batch * seq
    num_rows = table.shape[0]
    indices = x.T.reshape((num_indices,)).astype(jnp.int32)

    rows_per_worker = num_indices // NUM_WORKERS
    chunks = rows_per_worker // WINDOW
    outer = chunks // NBUF

    mesh = plsc.VectorSubcoreMesh(core_axis_name="c", subcore_axis_name="s")

    @pl.kernel(
        out_type=jax.ShapeDtypeStruct((seq, EMBED_DIM, batch), table.dtype),
        mesh=mesh,
        compiler_params=pltpu.CompilerParams(use_tc_tiling_on_sc=False,
                                             needs_layout_passes=False),
        scratch_types=[
            pltpu.VMEM((NBUF, WINDOW), jnp.int32),
            pltpu.VMEM((NBUF, WINDOW, EMBED_DIM), jnp.float32),
            pltpu.VMEM((NBUF, EMBED_DIM, WINDOW), jnp.float32),
            pltpu.SemaphoreType.DMA((NBUF,)),
            pltpu.SemaphoreType.DMA((NBUF,)),
            pltpu.SemaphoreType.DMA((NBUF,)),
        ],
    )
    def gather_kernel(table_hbm, i_hbm, o_hbm, idx_v, rows_v, trans_v,
                      sem_i, sem_g, sem_o):
        wid = lax.axis_index("s") * 2 + lax.axis_index("c")
        wbase = wid * rows_per_worker

        def start_idx(b, c):
            pltpu.async_copy(i_hbm.at[pl.ds(wbase + c * WINDOW, WINDOW)],
                             idx_v.at[b], sem_i.at[b])

        def wait_idx(b):
            pltpu.make_async_copy(i_hbm.at[pl.ds(0, WINDOW)],
                                  idx_v.at[b], sem_i.at[b]).wait()

        def start_gather(b):
            pltpu.async_copy(table_hbm.at[idx_v.at[b]], rows_v.at[b], sem_g.at[b])

        def wait_gather(b):
            pltpu.make_async_copy(table_hbm.at[idx_v.at[b]],
                                  rows_v.at[b], sem_g.at[b]).wait()

        def start_out(b, c):
            flat = wbase + c * WINDOW
            s = flat // batch
            i0 = flat % batch
            pltpu.async_copy(trans_v.at[b],
                             o_hbm.at[s, :, pl.ds(i0, WINDOW)],
                             sem_o.at[b])

        def wait_out(b):
            pltpu.make_async_copy(trans_v.at[b],
                                  o_hbm.at[0, :, pl.ds(0, WINDOW)],
                                  sem_o.at[b]).wait()

        def transpose(b):
            @pl.loop(0, WINDOW, step=16)
            def _(j0):
                row_ids = j0 + lax.iota(jnp.int32, 16)
                for f in range(EMBED_DIM):
                    col_ids = jnp.full((16,), f, jnp.int32)
                    vec = plsc.load_gather(rows_v.at[b], [row_ids, col_ids])
                    trans_v[b, f, pl.ds(j0, 16)] = vec

        for b in range(NBUF):
            start_idx(b, b)
        for b in range(NBUF):
            wait_idx(b)
            start_gather(b)

        @pl.loop(0, outer - 1)
        def _(o):
            cbase = o * NBUF
            for b in range(NBUF):
                wait_gather(b)
                transpose(b)
                start_out(b, cbase + b)
                start_idx(b, cbase + NBUF + b)
            for b in range(NBUF):
                wait_out(b)
                wait_idx(b)
                start_gather(b)

        for b in range(NBUF):
            wait_gather(b)
            transpose(b)
            start_out(b, (outer - 1) * NBUF + b)
        for b in range(NBUF):
            wait_out(b)

    out = gather_kernel(table, indices)
    return out.transpose(2, 0, 1)


# speedup vs baseline: 5.4717x; 1.3514x over previous
import jax
import jax.numpy as jnp
from jax import lax
from jax.experimental import pallas as pl
from jax.experimental.pallas import tpu as pltpu
from jax.experimental.pallas import tpu_sc as plsc

EMBED_DIM = 16
PACK = 128 // EMBED_DIM
NUM_WORKERS = 32
WINDOW = 512
NBUF = 4


def kernel(x, table):
    batch, seq = x.shape
    num_indices = batch * seq
    num_rows = table.shape[0]
    indices = x.T.reshape((num_indices,)).astype(jnp.int32)

    rows_per_worker = num_indices // NUM_WORKERS
    chunks = rows_per_worker // WINDOW
    outer = chunks // NBUF

    mesh = plsc.VectorSubcoreMesh(core_axis_name="c", subcore_axis_name="s")

    @pl.kernel(
        out_type=jax.ShapeDtypeStruct((seq, EMBED_DIM, batch), table.dtype),
        mesh=mesh,
        compiler_params=pltpu.CompilerParams(use_tc_tiling_on_sc=False,
                                             needs_layout_passes=False),
        scratch_types=[
            pltpu.VMEM((NBUF, WINDOW), jnp.int32),
            pltpu.VMEM((NBUF, WINDOW, EMBED_DIM), jnp.float32),
            pltpu.VMEM((NBUF, EMBED_DIM, WINDOW + 1), jnp.float32),
            pltpu.SemaphoreType.DMA((NBUF,)),
            pltpu.SemaphoreType.DMA((NBUF,)),
            pltpu.SemaphoreType.DMA((NBUF,)),
        ],
    )
    def gather_kernel(table_hbm, i_hbm, o_hbm, idx_v, rows_v, trans_v,
                      sem_i, sem_g, sem_o):
        wid = lax.axis_index("s") * 2 + lax.axis_index("c")
        wbase = wid * rows_per_worker

        def start_idx(b, c):
            pltpu.async_copy(i_hbm.at[pl.ds(wbase + c * WINDOW, WINDOW)],
                             idx_v.at[b], sem_i.at[b])

        def wait_idx(b):
            pltpu.make_async_copy(i_hbm.at[pl.ds(0, WINDOW)],
                                  idx_v.at[b], sem_i.at[b]).wait()

        def start_gather(b):
            pltpu.async_copy(table_hbm.at[idx_v.at[b]], rows_v.at[b], sem_g.at[b])

        def wait_gather(b):
            pltpu.make_async_copy(table_hbm.at[idx_v.at[b]],
                                  rows_v.at[b], sem_g.at[b]).wait()

        def start_out(b, c):
            flat = wbase + c * WINDOW
            s = flat // batch
            i0 = flat % batch
            pltpu.async_copy(trans_v.at[b, :, pl.ds(0, WINDOW)],
                             o_hbm.at[s, :, pl.ds(i0, WINDOW)],
                             sem_o.at[b])

        def wait_out(b):
            pltpu.make_async_copy(trans_v.at[b, :, pl.ds(0, WINDOW)],
                                  o_hbm.at[0, :, pl.ds(0, WINDOW)],
                                  sem_o.at[b]).wait()

        def transpose(b):
            f_ids = lax.iota(jnp.int32, 16)

            @pl.loop(0, WINDOW, step=16)
            def _(j0):
                for dj in range(16):
                    vec = rows_v[b, j0 + dj, :]
                    col_ids = jnp.full((16,), 0, jnp.int32) + (j0 + dj)
                    plsc.store_scatter(trans_v.at[b], [f_ids, col_ids], vec)

        for b in range(NBUF):
            start_idx(b, b)
        for b in range(NBUF):
            wait_idx(b)
            start_gather(b)

        @pl.loop(0, outer - 1)
        def _(o):
            cbase = o * NBUF
            for b in range(NBUF):
                wait_gather(b)
                transpose(b)
                start_out(b, cbase + b)
                start_idx(b, cbase + NBUF + b)
            for b in range(NBUF):
                wait_out(b)
                wait_idx(b)
                start_gather(b)

        for b in range(NBUF):
            wait_gather(b)
            transpose(b)
            start_out(b, (outer - 1) * NBUF + b)
        for b in range(NBUF):
            wait_out(b)

    out = gather_kernel(table, indices)
    return out.transpose(2, 0, 1)
